# Initial kernel scaffold; baseline (speedup 1.0000x reference)
import jax
import jax.numpy as jnp
from jax import lax
from jax.experimental import pallas as pl
from jax.experimental.pallas import tpu as pltpu
from jax.experimental.pallas import tpu_sc as plsc

_NC = 2
_NS = 16
_W = 32


def _make_spmm(nchunks, n_nodes, n_edges):
    rows_per_sub = n_nodes // _NS
    zr = 125
    edges_per_sub = n_edges // _NS
    eb = 80
    nb = edges_per_sub // eb

    mesh = plsc.VectorSubcoreMesh(core_axis_name="c", subcore_axis_name="s")
    out_type = [jax.ShapeDtypeStruct((n_nodes, _W), jnp.float32)] * nchunks
    scratch = [
        pltpu.VMEM((eb,), jnp.int32),
        pltpu.VMEM((eb,), jnp.int32),
        pltpu.VMEM((eb,), jnp.float32),
        pltpu.VMEM((eb, _W), jnp.float32),
        pltpu.VMEM((zr, _W), jnp.float32),
        pltpu.VMEM_SHARED((n_nodes, _W), jnp.float32),
    ]

    def body(*refs):
        x_refs = refs[:nchunks]
        col_ref, row_ref, val_ref = refs[nchunks:nchunks + 3]
        y_refs = refs[nchunks + 3:2 * nchunks + 3]
        cidx, ridx, vv, rows, zbuf, acc = refs[2 * nchunks + 3:]

        cid = lax.axis_index("c")
        sid = lax.axis_index("s")
        rbase = sid * rows_per_sub
        ebase = sid * edges_per_sub

        def zfill(k, carry):
            zbuf[k, pl.ds(0, 16)] = jnp.zeros((16,), jnp.float32)
            zbuf[k, pl.ds(16, 16)] = jnp.zeros((16,), jnp.float32)
            return carry
        lax.fori_loop(0, zr, zfill, None)

        for c in range(nchunks):
            @pl.when(cid == (c % _NC))
            def _process_chunk(c=c):
                def zslice(t, carry):
                    pltpu.sync_copy(zbuf, acc.at[pl.ds(rbase + t * zr, zr)])
                    return carry
                lax.fori_loop(0, rows_per_sub // zr, zslice, None)
                plsc.subcore_barrier()

                def batch(i, carry):
                    off = ebase + i * eb
                    pltpu.sync_copy(col_ref.at[pl.ds(off, eb)], cidx)
                    pltpu.sync_copy(row_ref.at[pl.ds(off, eb)], ridx)
                    pltpu.sync_copy(val_ref.at[pl.ds(off, eb)], vv)
                    pltpu.sync_copy(x_refs[c].at[cidx], rows)

                    def scale(k, icarry):
                        for u in range(8):
                            e = k * 8 + u
                            sv = jnp.full((16,), vv[e], jnp.float32)
                            for j in range(2):
                                sl = pl.ds(j * 16, 16)
                                rows[e, sl] = rows[e, sl] * sv
                        return icarry
                    lax.fori_loop(0, eb // 8, scale, None)

                    pltpu.sync_copy(rows, acc.at[ridx], add=True)
                    return carry
                lax.fori_loop(0, nb, batch, None)
                plsc.subcore_barrier()

                pltpu.sync_copy(acc.at[pl.ds(rbase, rows_per_sub)],
                                y_refs[c].at[pl.ds(rbase, rows_per_sub)])
        return None

    return pl.kernel(body, out_type=out_type, mesh=mesh, scratch_types=scratch)


def _make_decoder(n_nodes, n_edges):
    per_tile = n_edges // (_NC * _NS)
    eb = 128
    nb = -(-per_tile // eb)
    last_off = per_tile - eb

    mesh = plsc.VectorSubcoreMesh(core_axis_name="c", subcore_axis_name="s")
    out_type = jax.ShapeDtypeStruct((n_edges,), jnp.float32)
    scratch = [
        pltpu.VMEM((eb,), jnp.int32),
        pltpu.VMEM((eb,), jnp.int32),
        pltpu.VMEM((eb, _W), jnp.float32),
        pltpu.VMEM((eb, _W), jnp.float32),
        pltpu.VMEM((eb,), jnp.float32),
    ]

    def body(z_ref, src_ref, dst_ref, out_ref, si, di, srow, drow, ov):
        cid = lax.axis_index("c")
        sid = lax.axis_index("s")
        base = (sid * _NC + cid) * per_tile

        def batch(i, carry):
            off = base + jnp.minimum(i * eb, last_off)
            pltpu.sync_copy(src_ref.at[pl.ds(off, eb)], si)
            pltpu.sync_copy(dst_ref.at[pl.ds(off, eb)], di)
            pltpu.sync_copy(z_ref.at[si], srow)
            pltpu.sync_copy(z_ref.at[di], drow)

            def dots(k, icarry):
                for u in range(8):
                    e = k * 8 + u
                    a0 = srow[e, pl.ds(0, 16)]
                    a1 = srow[e, pl.ds(16, 16)]
                    b0 = drow[e, pl.ds(0, 16)]
                    b1 = drow[e, pl.ds(16, 16)]
                    ov[e] = jnp.sum(a0 * b0 + a1 * b1)
                return icarry
            lax.fori_loop(0, eb // 8, dots, None)

            pltpu.sync_copy(ov, out_ref.at[pl.ds(off, eb)])
            return carry
        lax.fori_loop(0, nb, batch, None)
        return None

    return pl.kernel(body, out_type=out_type, mesh=mesh, scratch_types=scratch)


def _encode_tc(ys, w1p, b1p, w2p, w1f, b1f, w2f, w1n, b1n, w2n):
    n = ys[0].shape[0]
    r = 1000

    def body(y0, y1, y2, y3, y4, w1pr, b1pr, w2pr, w1fr, b1fr, w2fr,
             w1nr, b1nr, w2nr, o0, o1, o2):
        y = jnp.concatenate([y0[...], y1[...], y2[...], y3[...], y4[...]],
                            axis=1)
        hp = jnp.maximum(
            jnp.dot(y[:, 0:64], w1pr[...], preferred_element_type=jnp.float32)
            + b1pr[...], 0.0)
        o0[...] = jnp.dot(hp, w2pr[...], preferred_element_type=jnp.float32)
        hf = jnp.maximum(
            jnp.dot(y[:, 64:112], w1fr[...], preferred_element_type=jnp.float32)
            + b1fr[...], 0.0)
        o1[...] = jnp.dot(hf, w2fr[...], preferred_element_type=jnp.float32)
        hn = jnp.maximum(
            jnp.dot(y[:, 112:160], w1nr[...], preferred_element_type=jnp.float32)
            + b1nr[...], 0.0)
        o2[...] = jnp.dot(hn, w2nr[...], preferred_element_type=jnp.float32)

    row_spec = pl.BlockSpec((r, _W), lambda i: (i, 0))
    full = lambda shape: pl.BlockSpec(shape, lambda i: (0,) * len(shape))
    return pl.pallas_call(
        body,
        grid=(n // r,),
        in_specs=[row_spec] * 5 + [
            full((64, 64)), full((1, 64)), full((64, 32)),
            full((48, 64)), full((1, 64)), full((48, 32)),
            full((48, 64)), full((1, 64)), full((48, 32)),
        ],
        out_specs=[row_spec] * 3,
        out_shape=[jax.ShapeDtypeStruct((n, _W), jnp.float32)] * 3,
    )(*ys, w1p, b1p, w2p, w1f, b1f, w2f, w1n, b1n, w2n)


def _gate_tc(z0, z1, z2, b2p, b2f, b2n, gwp, gwf, gwn, gbp, gbf, gbn):
    n = z0.shape[0]
    r = 1000

    def body(z0r, z1r, z2r, b2pr, b2fr, b2nr, gwpr, gwfr, gwnr,
             gbpr, gbfr, gbnr, out):
        zp = z0r[...] + b2pr[...]
        zf = z1r[...] + b2fr[...]
        zn = z2r[...] + b2nr[...]
        sp = jnp.sum(zp * gwpr[...], axis=1, keepdims=True) + gbpr[...]
        sf = jnp.sum(zf * gwfr[...], axis=1, keepdims=True) + gbfr[...]
        sn = jnp.sum(zn * gwnr[...], axis=1, keepdims=True) + gbnr[...]
        s = jnp.concatenate([sp, sf, sn], axis=1)
        m = jnp.max(s, axis=1, keepdims=True)
        e = jnp.exp(s - m)
        a = e / jnp.sum(e, axis=1, keepdims=True)
        out[...] = a[:, 0:1] * zp + a[:, 1:2] * zf + a[:, 2:3] * zn

    row_spec = pl.BlockSpec((r, _W), lambda i: (i, 0))
    full = lambda shape: pl.BlockSpec(shape, lambda i: (0,) * len(shape))
    return pl.pallas_call(
        body,
        grid=(n // r,),
        in_specs=[row_spec] * 3 + [full((1, _W))] * 6 + [full((1, 1))] * 3,
        out_specs=row_spec,
        out_shape=jax.ShapeDtypeStruct((n, _W), jnp.float32),
    )(z0, z1, z2, b2p, b2f, b2n, gwp, gwf, gwn, gbp, gbf, gbn)


def _pad_cols(a, w):
    return jnp.pad(a, ((0, 0), (0, w - a.shape[1])))


def _pad_rows(a, w):
    return jnp.pad(a, ((0, w - a.shape[0]), (0, 0)))


def kernel(process_view, file_view, network_view, adj_indices, adj_values,
           edge_index,
           process_W1, process_b1, process_W2, process_b2, process_gW,
           process_gb,
           file_W1, file_b1, file_W2, file_b2, file_gW, file_gb,
           network_W1, network_b1, network_W2, network_b2, network_gW,
           network_gb):
    n = process_view.shape[0]
    n_edges = adj_values.shape[0]

    row = adj_indices[0].astype(jnp.int32)
    col = adj_indices[1].astype(jnp.int32)
    val = adj_values.astype(jnp.float32)

    x = jnp.concatenate([_pad_cols(process_view, 64),
                         _pad_cols(file_view, 48),
                         _pad_cols(network_view, 48)], axis=1)
    xs = tuple(x[:, _W * c:_W * (c + 1)] for c in range(5))

    ys = _make_spmm(5, n, n_edges)(*xs, col, row, val)

    ps = _encode_tc(
        ys,
        _pad_rows(process_W1, 64), process_b1.reshape(1, 64), process_W2,
        _pad_rows(file_W1, 48), file_b1.reshape(1, 64), file_W2,
        _pad_rows(network_W1, 48), network_b1.reshape(1, 64), network_W2,
    )

    zs = _make_spmm(3, n, n_edges)(*ps, col, row, val)

    z_fused = _gate_tc(
        zs[0], zs[1], zs[2],
        process_b2.reshape(1, _W), file_b2.reshape(1, _W),
        network_b2.reshape(1, _W),
        process_gW.reshape(1, _W), file_gW.reshape(1, _W),
        network_gW.reshape(1, _W),
        process_gb.reshape(1, 1), file_gb.reshape(1, 1),
        network_gb.reshape(1, 1),
    )

    src = edge_index[0].astype(jnp.int32)
    dst = edge_index[1].astype(jnp.int32)
    return _make_decoder(n, n_edges)(z_fused, src, dst)


# speedup vs baseline: 2.4951x; 2.4951x over previous
import jax
import jax.numpy as jnp
from jax import lax
from jax.experimental import pallas as pl
from jax.experimental.pallas import tpu as pltpu
from jax.experimental.pallas import tpu_sc as plsc

_NC = 2
_NS = 16
_W = 32


def _make_spmm(nchunks, n_nodes, n_edges):
    rows_per_sub = n_nodes // _NS
    wrows = -8 * (-n_nodes // (8 * _NS))
    wtail = n_nodes - (_NS - 1) * wrows
    zr = 125
    edges_per_sub = n_edges // _NS
    eb = 80
    nb = edges_per_sub // eb

    mesh = plsc.VectorSubcoreMesh(core_axis_name="c", subcore_axis_name="s")
    out_type = [jax.ShapeDtypeStruct((n_nodes, _W), jnp.float32)] * nchunks
    scratch = [
        pltpu.VMEM((eb,), jnp.int32),
        pltpu.VMEM((eb,), jnp.int32),
        pltpu.VMEM((eb,), jnp.float32),
        pltpu.VMEM((eb, _W), jnp.float32),
        pltpu.VMEM((zr, _W), jnp.float32),
        pltpu.VMEM_SHARED((n_nodes, _W), jnp.float32),
    ]

    def body(*refs):
        x_refs = refs[:nchunks]
        col_ref, row_ref, val_ref = refs[nchunks:nchunks + 3]
        y_refs = refs[nchunks + 3:2 * nchunks + 3]
        cidx, ridx, vv, rows, zbuf, acc = refs[2 * nchunks + 3:]

        cid = lax.axis_index("c")
        sid = lax.axis_index("s")
        rbase = sid * rows_per_sub
        ebase = sid * edges_per_sub

        def zfill(k, carry):
            zbuf[k, pl.ds(0, 16)] = jnp.zeros((16,), jnp.float32)
            zbuf[k, pl.ds(16, 16)] = jnp.zeros((16,), jnp.float32)
            return carry
        lax.fori_loop(0, zr, zfill, None)

        for c in range(nchunks):
            @pl.when(cid == (c % _NC))
            def _process_chunk(c=c):
                def zslice(t, carry):
                    pltpu.sync_copy(zbuf, acc.at[pl.ds(rbase + t * zr, zr)])
                    return carry
                lax.fori_loop(0, rows_per_sub // zr, zslice, None)
                plsc.subcore_barrier()

                def batch(i, carry):
                    off = ebase + i * eb
                    pltpu.sync_copy(col_ref.at[pl.ds(off, eb)], cidx)
                    pltpu.sync_copy(row_ref.at[pl.ds(off, eb)], ridx)
                    pltpu.sync_copy(val_ref.at[pl.ds(off, eb)], vv)
                    pltpu.sync_copy(x_refs[c].at[cidx], rows)

                    def scale(k, icarry):
                        val16 = vv[pl.ds(k * 16, 16)]
                        for u in range(16):
                            e = k * 16 + u
                            sv = jnp.full((16,), val16[u], jnp.float32)
                            for j in range(2):
                                sl = pl.ds(j * 16, 16)
                                rows[e, sl] = rows[e, sl] * sv
                        return icarry
                    lax.fori_loop(0, eb // 16, scale, None)

                    pltpu.sync_copy(rows, acc.at[ridx], add=True)
                    return carry
                lax.fori_loop(0, nb, batch, None)
                plsc.subcore_barrier()

                wbase = sid * wrows
                @pl.when(sid < _NS - 1)
                def _write_main():
                    pltpu.sync_copy(acc.at[pl.ds(wbase, wrows)],
                                    y_refs[c].at[pl.ds(wbase, wrows)])
                @pl.when(sid == _NS - 1)
                def _write_tail():
                    pltpu.sync_copy(acc.at[pl.ds(wbase, wtail)],
                                    y_refs[c].at[pl.ds(wbase, wtail)])
        return None

    return pl.kernel(body, out_type=out_type, mesh=mesh, scratch_types=scratch,
                     compiler_params=pltpu.CompilerParams(
                         use_tc_tiling_on_sc=False,
                         needs_layout_passes=False))


def _make_decoder(n_nodes, n_edges):
    per_tile = n_edges // (_NC * _NS)
    eb = 128
    nb = -(-per_tile // eb)
    last_off = per_tile - eb

    mesh = plsc.VectorSubcoreMesh(core_axis_name="c", subcore_axis_name="s")
    out_type = jax.ShapeDtypeStruct((n_edges,), jnp.float32)
    scratch = [
        pltpu.VMEM((eb,), jnp.int32),
        pltpu.VMEM((eb,), jnp.int32),
        pltpu.VMEM((eb, _W), jnp.float32),
        pltpu.VMEM((eb, _W), jnp.float32),
        pltpu.VMEM((eb,), jnp.float32),
    ]

    def body(z_ref, src_ref, dst_ref, out_ref, si, di, srow, drow, ov):
        cid = lax.axis_index("c")
        sid = lax.axis_index("s")
        base = (sid * _NC + cid) * per_tile

        def batch(i, carry):
            off = base + jnp.minimum(i * eb, last_off)
            pltpu.sync_copy(src_ref.at[pl.ds(off, eb)], si)
            pltpu.sync_copy(dst_ref.at[pl.ds(off, eb)], di)
            pltpu.sync_copy(z_ref.at[si], srow)
            pltpu.sync_copy(z_ref.at[di], drow)

            def dots(k, icarry):
                e0 = k * 16
                re = e0 + lax.iota(jnp.int32, 16)
                acc = jnp.zeros((16,), jnp.float32)
                for j in range(_W):
                    fj = jnp.full((16,), j, jnp.int32)
                    acc = acc + (plsc.load_gather(srow, [re, fj])
                                 * plsc.load_gather(drow, [re, fj]))
                ov[pl.ds(e0, 16)] = acc
                return icarry
            lax.fori_loop(0, eb // 16, dots, None)

            pltpu.sync_copy(ov, out_ref.at[pl.ds(off, eb)])
            return carry
        lax.fori_loop(0, nb, batch, None)
        return None

    return pl.kernel(body, out_type=out_type, mesh=mesh, scratch_types=scratch,
                     compiler_params=pltpu.CompilerParams(
                         use_tc_tiling_on_sc=False,
                         needs_layout_passes=False))


def _encode_tc(ys, w1p, b1p, w2p, w1f, b1f, w2f, w1n, b1n, w2n):
    n = ys[0].shape[0]
    r = 1000

    def body(y0, y1, y2, y3, y4, w1pr, b1pr, w2pr, w1fr, b1fr, w2fr,
             w1nr, b1nr, w2nr, o0, o1, o2):
        y = jnp.concatenate([y0[...], y1[...], y2[...], y3[...], y4[...]],
                            axis=1)
        hp = jnp.maximum(
            jnp.dot(y[:, 0:64], w1pr[...], preferred_element_type=jnp.float32)
            + b1pr[...], 0.0)
        o0[...] = jnp.dot(hp, w2pr[...], preferred_element_type=jnp.float32)
        hf = jnp.maximum(
            jnp.dot(y[:, 64:112], w1fr[...], preferred_element_type=jnp.float32)
            + b1fr[...], 0.0)
        o1[...] = jnp.dot(hf, w2fr[...], preferred_element_type=jnp.float32)
        hn = jnp.maximum(
            jnp.dot(y[:, 112:160], w1nr[...], preferred_element_type=jnp.float32)
            + b1nr[...], 0.0)
        o2[...] = jnp.dot(hn, w2nr[...], preferred_element_type=jnp.float32)

    row_spec = pl.BlockSpec((r, _W), lambda i: (i, 0))
    full = lambda shape: pl.BlockSpec(shape, lambda i: (0,) * len(shape))
    return pl.pallas_call(
        body,
        grid=(n // r,),
        in_specs=[row_spec] * 5 + [
            full((64, 64)), full((1, 64)), full((64, 32)),
            full((48, 64)), full((1, 64)), full((64, 32)),
            full((48, 64)), full((1, 64)), full((64, 32)),
        ],
        out_specs=[row_spec] * 3,
        out_shape=[jax.ShapeDtypeStruct((n, _W), jnp.float32)] * 3,
    )(*ys, w1p, b1p, w2p, w1f, b1f, w2f, w1n, b1n, w2n)


def _gate_tc(z0, z1, z2, b2p, b2f, b2n, gwp, gwf, gwn, gbp, gbf, gbn):
    n = z0.shape[0]
    r = 1000

    def body(z0r, z1r, z2r, b2pr, b2fr, b2nr, gwpr, gwfr, gwnr,
             gbpr, gbfr, gbnr, out):
        zp = z0r[...] + b2pr[...]
        zf = z1r[...] + b2fr[...]
        zn = z2r[...] + b2nr[...]
        sp = jnp.sum(zp * gwpr[...], axis=1, keepdims=True) + gbpr[...]
        sf = jnp.sum(zf * gwfr[...], axis=1, keepdims=True) + gbfr[...]
        sn = jnp.sum(zn * gwnr[...], axis=1, keepdims=True) + gbnr[...]
        s = jnp.concatenate([sp, sf, sn], axis=1)
        m = jnp.max(s, axis=1, keepdims=True)
        e = jnp.exp(s - m)
        a = e / jnp.sum(e, axis=1, keepdims=True)
        out[...] = a[:, 0:1] * zp + a[:, 1:2] * zf + a[:, 2:3] * zn

    row_spec = pl.BlockSpec((r, _W), lambda i: (i, 0))
    full = lambda shape: pl.BlockSpec(shape, lambda i: (0,) * len(shape))
    return pl.pallas_call(
        body,
        grid=(n // r,),
        in_specs=[row_spec] * 3 + [full((1, _W))] * 6 + [full((1, 1))] * 3,
        out_specs=row_spec,
        out_shape=jax.ShapeDtypeStruct((n, _W), jnp.float32),
    )(z0, z1, z2, b2p, b2f, b2n, gwp, gwf, gwn, gbp, gbf, gbn)


def _pad_cols(a, w):
    return jnp.pad(a, ((0, 0), (0, w - a.shape[1])))


def _pad_rows(a, w):
    return jnp.pad(a, ((0, w - a.shape[0]), (0, 0)))


def kernel(process_view, file_view, network_view, adj_indices, adj_values,
           edge_index,
           process_W1, process_b1, process_W2, process_b2, process_gW,
           process_gb,
           file_W1, file_b1, file_W2, file_b2, file_gW, file_gb,
           network_W1, network_b1, network_W2, network_b2, network_gW,
           network_gb):
    n = process_view.shape[0]
    n_edges = adj_values.shape[0]

    row = adj_indices[0].astype(jnp.int32)
    col = adj_indices[1].astype(jnp.int32)
    val = adj_values.astype(jnp.float32)

    x = jnp.concatenate([_pad_cols(process_view, 64),
                         _pad_cols(file_view, 48),
                         _pad_cols(network_view, 48)], axis=1)
    xs = tuple(x[:, _W * c:_W * (c + 1)] for c in range(5))

    ys = _make_spmm(5, n, n_edges)(*xs, col, row, val)

    ps = _encode_tc(
        ys,
        _pad_rows(process_W1, 64), process_b1.reshape(1, 64), process_W2,
        _pad_rows(file_W1, 48), file_b1.reshape(1, 64), file_W2,
        _pad_rows(network_W1, 48), network_b1.reshape(1, 64), network_W2,
    )

    zs = _make_spmm(3, n, n_edges)(*ps, col, row, val)

    z_fused = _gate_tc(
        zs[0], zs[1], zs[2],
        process_b2.reshape(1, _W), file_b2.reshape(1, _W),
        network_b2.reshape(1, _W),
        process_gW.reshape(1, _W), file_gW.reshape(1, _W),
        network_gW.reshape(1, _W),
        process_gb.reshape(1, 1), file_gb.reshape(1, 1),
        network_gb.reshape(1, 1),
    )

    src = edge_index[0].astype(jnp.int32)
    dst = edge_index[1].astype(jnp.int32)
    return _make_decoder(n, n_edges)(z_fused, src, dst)


# speedup vs baseline: 5.9634x; 2.3901x over previous
import jax
import jax.numpy as jnp
from jax import lax
from jax.experimental import pallas as pl
from jax.experimental.pallas import tpu as pltpu
from jax.experimental.pallas import tpu_sc as plsc

_NC = 2
_NS = 16
_W = 32
_EB = 80
_SBB = 5
_SB = _EB * _SBB

_SC_PARAMS = pltpu.CompilerParams(use_tc_tiling_on_sc=False,
                                  needs_layout_passes=False)


def _make_spmm(nchunks, n_nodes, n_edges):
    rows_per_sub = n_nodes // _NS
    wrows = -8 * (-n_nodes // (8 * _NS))
    wtail = n_nodes - (_NS - 1) * wrows
    sbs_per_sub = n_edges // (_NS * _SB)
    split_lo = sbs_per_sub // 2

    mesh = plsc.VectorSubcoreMesh(core_axis_name="c", subcore_axis_name="s")
    split = nchunks % 2 == 1
    n_out = nchunks + 1 if split else nchunks
    n_full = nchunks - 1 if split else nchunks
    out_type = [jax.ShapeDtypeStruct((n_nodes, _W), jnp.float32)] * n_out
    scratch = [
        pltpu.VMEM((_SBB, _EB), jnp.int32),
        pltpu.VMEM((_SBB, _EB), jnp.int32),
        pltpu.VMEM((_SBB, _EB), jnp.float32),
        pltpu.VMEM((_SBB, _EB), jnp.int32),
        pltpu.VMEM((_SBB, _EB), jnp.int32),
        pltpu.VMEM((_SBB, _EB), jnp.float32),
        pltpu.VMEM((_SB, _W), jnp.float32),
        pltpu.VMEM_SHARED((n_nodes, _W), jnp.float32),
        pltpu.SemaphoreType.DMA,
        pltpu.SemaphoreType.DMA,
        pltpu.SemaphoreType.DMA,
    ]

    def body(*refs):
        x_refs = refs[:nchunks]
        col_ref, row_ref, val_ref, zeros_ref = refs[nchunks:nchunks + 4]
        y_refs = refs[nchunks + 4:nchunks + 4 + n_out]
        (ca, ra, va, cb, rb, vb, rows, acc,
         isem, gsem, ssem) = refs[nchunks + 4 + n_out:]

        cid = lax.axis_index("c")
        sid = lax.axis_index("s")
        rbase = sid * rows_per_sub
        sb_row0 = sid * (sbs_per_sub * _SBB)

        def stage(s, bufs):
            roff = sb_row0 + s * _SBB
            pltpu.async_copy(col_ref.at[pl.ds(roff, _SBB)], bufs[0], isem)
            pltpu.async_copy(row_ref.at[pl.ds(roff, _SBB)], bufs[1], isem)
            pltpu.async_copy(val_ref.at[pl.ds(roff, _SBB)], bufs[2], isem)

        def wait_stage():
            for _ in range(3):
                pltpu.make_async_copy(col_ref.at[pl.ds(0, _SBB)], ca,
                                      isem).wait()

        def process(x_ref, s, bufs, nxt, hi):
            wait_stage()
            @pl.when(s + 1 < hi)
            def _stage_next():
                stage(s + 1, nxt)
            cbuf, rbuf, vbuf = bufs

            def fire_g(j, carry):
                pltpu.async_copy(x_ref.at[cbuf.at[j]],
                                 rows.at[pl.ds(j * _EB, _EB)], gsem)
                return carry
            lax.fori_loop(0, _SBB, fire_g, None)

            def drain_g(j, carry):
                pltpu.make_async_copy(x_ref.at[cbuf.at[0]],
                                      rows.at[pl.ds(0, _EB)], gsem).wait()
                return carry
            lax.fori_loop(0, _SBB, drain_g, None)

            def scale(j, carry):
                def scale16(k, icarry):
                    val16 = vbuf[j, pl.ds(k * 16, 16)]
                    for u in range(16):
                        e = j * _EB + k * 16 + u
                        sv = jnp.full((16,), val16[u], jnp.float32)
                        for h in range(2):
                            sl = pl.ds(h * 16, 16)
                            rows[e, sl] = rows[e, sl] * sv
                    return icarry
                lax.fori_loop(0, _EB // 16, scale16, None)
                return carry
            lax.fori_loop(0, _SBB, scale, None)

            def fire_s(j, carry):
                pltpu.async_copy(rows.at[pl.ds(j * _EB, _EB)],
                                 acc.at[rbuf.at[j]], ssem, add=True)
                pltpu.make_async_copy(rows.at[pl.ds(0, _EB)],
                                      acc.at[rbuf.at[0]], ssem).wait()
                return carry
            lax.fori_loop(0, _SBB, fire_s, None)

        def chunk_pass(x_ref, y_ref, lo, hi):
            plsc.subcore_barrier()
            pltpu.sync_copy(zeros_ref, acc.at[pl.ds(rbase, rows_per_sub)])
            plsc.subcore_barrier()

            stage(lo, (ca, ra, va))

            def pair(t, carry):
                s0 = lo + 2 * t
                process(x_ref, s0, (ca, ra, va), (cb, rb, vb), hi)
                @pl.when(s0 + 1 < hi)
                def _odd():
                    process(x_ref, s0 + 1, (cb, rb, vb), (ca, ra, va), hi)
                return carry
            lax.fori_loop(0, (hi - lo + 1) // 2, pair, None)
            plsc.subcore_barrier()

            wbase = sid * wrows
            @pl.when(sid < _NS - 1)
            def _write_main():
                pltpu.sync_copy(acc.at[pl.ds(wbase, wrows)],
                                y_ref.at[pl.ds(wbase, wrows)])
            @pl.when(sid == _NS - 1)
            def _write_tail():
                pltpu.sync_copy(acc.at[pl.ds(wbase, wtail)],
                                y_ref.at[pl.ds(wbase, wtail)])

        for c in range(n_full):
            @pl.when(cid == (c % _NC))
            def _full_chunk(c=c):
                chunk_pass(x_refs[c], y_refs[c], 0, sbs_per_sub)

        if split:
            @pl.when(cid == 0)
            def _split_a():
                chunk_pass(x_refs[nchunks - 1], y_refs[nchunks - 1], 0,
                           split_lo)
            @pl.when(cid == 1)
            def _split_b():
                chunk_pass(x_refs[nchunks - 1], y_refs[nchunks], split_lo,
                           sbs_per_sub)
        return None

    return pl.kernel(body, out_type=out_type, mesh=mesh,
                     scratch_types=scratch, compiler_params=_SC_PARAMS)


def _make_decoder(n_nodes, n_edges):
    per_tile = n_edges // (_NC * _NS)
    eb = 256
    nb = -(-per_tile // eb)
    last_off = per_tile - eb

    mesh = plsc.VectorSubcoreMesh(core_axis_name="c", subcore_axis_name="s")
    out_type = jax.ShapeDtypeStruct((n_edges,), jnp.float32)
    scratch = [
        pltpu.VMEM((eb,), jnp.int32),
        pltpu.VMEM((eb,), jnp.int32),
        pltpu.VMEM((eb,), jnp.int32),
        pltpu.VMEM((eb,), jnp.int32),
        pltpu.VMEM((eb, _W), jnp.float32),
        pltpu.VMEM((eb, _W), jnp.float32),
        pltpu.VMEM((eb,), jnp.float32),
        pltpu.VMEM((eb,), jnp.float32),
        pltpu.SemaphoreType.DMA,
        pltpu.SemaphoreType.DMA,
        pltpu.SemaphoreType.DMA,
    ]

    def body(z_ref, src_ref, dst_ref, out_ref,
             sa, da, sb, db, srow, drow, ova, ovb, isem, gsem, osem):
        cid = lax.axis_index("c")
        sid = lax.axis_index("s")
        base = (sid * _NC + cid) * per_tile

        def off_of(i):
            return base + jnp.minimum(i * eb, last_off)

        def stage(i, bufs):
            off = off_of(i)
            pltpu.async_copy(src_ref.at[pl.ds(off, eb)], bufs[0], isem)
            pltpu.async_copy(dst_ref.at[pl.ds(off, eb)], bufs[1], isem)

        def process(i, bufs, nxt, drain_out):
            for _ in range(2):
                pltpu.make_async_copy(src_ref.at[pl.ds(base, eb)], sa,
                                      isem).wait()
            if isinstance(i, int):
                if i + 1 < nb:
                    stage(i + 1, nxt)
            else:
                @pl.when(i + 1 < nb)
                def _stage_next():
                    stage(i + 1, nxt)
            sidx, didx, ovbuf = bufs
            for half in range(2):
                hs = pl.ds(half * 128, 128)
                pltpu.async_copy(z_ref.at[sidx.at[hs]], srow.at[hs], gsem)
                pltpu.async_copy(z_ref.at[didx.at[hs]], drow.at[hs], gsem)
            for _ in range(4):
                pltpu.make_async_copy(z_ref.at[sa.at[pl.ds(0, 128)]],
                                      srow.at[pl.ds(0, 128)], gsem).wait()

            if drain_out:
                pltpu.make_async_copy(ova, out_ref.at[pl.ds(base, eb)],
                                      osem).wait()

            def dots(k, icarry):
                e0 = k * 16
                re = e0 + lax.iota(jnp.int32, 16)
                dacc = jnp.zeros((16,), jnp.float32)
                for j in range(_W):
                    fj = jnp.full((16,), j, jnp.int32)
                    dacc = dacc + (plsc.load_gather(srow, [re, fj])
                                   * plsc.load_gather(drow, [re, fj]))
                ovbuf[pl.ds(e0, 16)] = dacc
                return icarry
            lax.fori_loop(0, eb // 16, dots, None)

            pltpu.async_copy(ovbuf, out_ref.at[pl.ds(off_of(i), eb)], osem)

        stage(0, (sa, da))
        process(0, (sa, da, ova), (sb, db), False)
        process(1, (sb, db, ovb), (sa, da), False)

        def pairloop(t, carry):
            i0 = 2 * t
            process(i0, (sa, da, ova), (sb, db), True)
            process(i0 + 1, (sb, db, ovb), (sa, da), True)
            return carry
        lax.fori_loop(1, nb // 2, pairloop, None)
        for _ in range(2):
            pltpu.make_async_copy(ova, out_ref.at[pl.ds(base, eb)],
                                  osem).wait()
        return None

    return pl.kernel(body, out_type=out_type, mesh=mesh,
                     scratch_types=scratch, compiler_params=_SC_PARAMS)


def _encode_tc(ys, w1p, b1p, w1f, b1f, w1n, b1n):
    n = ys[0].shape[0]
    r = 1000

    def body(y0, y1, y2, y3, y4a, y4b, w1pr, b1pr, w1fr, b1fr,
             w1nr, b1nr, o0, o1, o2, o3, o4, o5):
        y = jnp.concatenate([y0[...], y1[...], y2[...], y3[...],
                             y4a[...] + y4b[...]], axis=1)
        hp = jnp.maximum(
            jnp.dot(y[:, 0:64], w1pr[...], preferred_element_type=jnp.float32)
            + b1pr[...], 0.0)
        o0[...] = hp[:, 0:32]
        o1[...] = hp[:, 32:64]
        hf = jnp.maximum(
            jnp.dot(y[:, 64:112], w1fr[...], preferred_element_type=jnp.float32)
            + b1fr[...], 0.0)
        o2[...] = hf[:, 0:32]
        o3[...] = hf[:, 32:64]
        hn = jnp.maximum(
            jnp.dot(y[:, 112:160], w1nr[...], preferred_element_type=jnp.float32)
            + b1nr[...], 0.0)
        o4[...] = hn[:, 0:32]
        o5[...] = hn[:, 32:64]

    row_spec = pl.BlockSpec((r, _W), lambda i: (i, 0))
    full = lambda shape: pl.BlockSpec(shape, lambda i: (0,) * len(shape))
    return pl.pallas_call(
        body,
        grid=(n // r,),
        in_specs=[row_spec] * 6 + [
            full((64, 64)), full((1, 64)),
            full((48, 64)), full((1, 64)),
            full((48, 64)), full((1, 64)),
        ],
        out_specs=[row_spec] * 6,
        out_shape=[jax.ShapeDtypeStruct((n, _W), jnp.float32)] * 6,
    )(*ys, w1p, b1p, w1f, b1f, w1n, b1n)


def _gate_tc(hs, w2p, w2f, w2n, b2p, b2f, b2n, gwp, gwf, gwn,
             gbp, gbf, gbn):
    n = hs[0].shape[0]
    r = 1000

    def body(h0, h1, h2, h3, h4, h5, w2pr, w2fr, w2nr, b2pr, b2fr, b2nr,
             gwpr, gwfr, gwnr, gbpr, gbfr, gbnr, out):
        ap = jnp.concatenate([h0[...], h1[...]], axis=1)
        af = jnp.concatenate([h2[...], h3[...]], axis=1)
        an = jnp.concatenate([h4[...], h5[...]], axis=1)
        zp = jnp.dot(ap, w2pr[...], preferred_element_type=jnp.float32) \
            + b2pr[...]
        zf = jnp.dot(af, w2fr[...], preferred_element_type=jnp.float32) \
            + b2fr[...]
        zn = jnp.dot(an, w2nr[...], preferred_element_type=jnp.float32) \
            + b2nr[...]
        sp = jnp.dot(zp, gwpr[...], preferred_element_type=jnp.float32) \
            + gbpr[...]
        sf = jnp.dot(zf, gwfr[...], preferred_element_type=jnp.float32) \
            + gbfr[...]
        sn = jnp.dot(zn, gwnr[...], preferred_element_type=jnp.float32) \
            + gbnr[...]
        s = jnp.concatenate([sp, sf, sn], axis=1)
        m = jnp.max(s, axis=1, keepdims=True)
        e = jnp.exp(s - m)
        a = e / jnp.sum(e, axis=1, keepdims=True)
        out[...] = a[:, 0:1] * zp + a[:, 1:2] * zf + a[:, 2:3] * zn

    row_spec = pl.BlockSpec((r, _W), lambda i: (i, 0))
    full = lambda shape: pl.BlockSpec(shape, lambda i: (0,) * len(shape))
    return pl.pallas_call(
        body,
        grid=(n // r,),
        in_specs=[row_spec] * 6 + [full((64, 32))] * 3 + [full((1, _W))] * 3
        + [full((32, 1))] * 3 + [full((1, 1))] * 3,
        out_specs=row_spec,
        out_shape=jax.ShapeDtypeStruct((n, _W), jnp.float32),
    )(*hs, w2p, w2f, w2n, b2p, b2f, b2n, gwp, gwf, gwn, gbp, gbf, gbn)


def _pad_cols(a, w):
    return jnp.pad(a, ((0, 0), (0, w - a.shape[1])))


def _pad_rows(a, w):
    return jnp.pad(a, ((0, w - a.shape[0]), (0, 0)))


def kernel(process_view, file_view, network_view, adj_indices, adj_values,
           edge_index,
           process_W1, process_b1, process_W2, process_b2, process_gW,
           process_gb,
           file_W1, file_b1, file_W2, file_b2, file_gW, file_gb,
           network_W1, network_b1, network_W2, network_b2, network_gW,
           network_gb):
    n = process_view.shape[0]
    n_edges = adj_values.shape[0]

    row = adj_indices[0].astype(jnp.int32).reshape(n_edges // _EB, _EB)
    col = adj_indices[1].astype(jnp.int32).reshape(n_edges // _EB, _EB)
    val = adj_values.astype(jnp.float32).reshape(n_edges // _EB, _EB)

    x = jnp.concatenate([_pad_cols(process_view, 64),
                         _pad_cols(file_view, 48),
                         _pad_cols(network_view, 48)], axis=1)
    xs = tuple(x[:, _W * c:_W * (c + 1)] for c in range(5))

    zeros = jnp.zeros((n // _NS, _W), jnp.float32)
    ys = _make_spmm(5, n, n_edges)(*xs, col, row, val, zeros)

    ps = _encode_tc(
        ys,
        _pad_rows(process_W1, 64), process_b1.reshape(1, 64),
        _pad_rows(file_W1, 48), file_b1.reshape(1, 64),
        _pad_rows(network_W1, 48), network_b1.reshape(1, 64),
    )

    zs = _make_spmm(6, n, n_edges)(*ps, col, row, val, zeros)

    z_fused = _gate_tc(
        zs, process_W2, file_W2, network_W2,
        process_b2.reshape(1, _W), file_b2.reshape(1, _W),
        network_b2.reshape(1, _W),
        process_gW, file_gW, network_gW,
        process_gb.reshape(1, 1), file_gb.reshape(1, 1),
        network_gb.reshape(1, 1),
    )

    src = edge_index[0].astype(jnp.int32)
    dst = edge_index[1].astype(jnp.int32)
    return _make_decoder(n, n_edges)(z_fused, src, dst)
